# Initial kernel scaffold; baseline (speedup 1.0000x reference)
import jax
import jax.numpy as jnp
from jax.experimental import pallas as pl


def kernel(feat_index, feat_value, em1_table, em2_table):
    raise NotImplementedError("write your pallas kernel here")


# speedup vs baseline: 1.9640x; 1.9640x over previous
import functools

import jax
import jax.numpy as jnp
from jax import lax
from jax.experimental import pallas as pl
from jax.experimental.pallas import tpu as pltpu
from jax.experimental.pallas import tpu_sc as plsc

_B, _F, _V, _D = 16384, 26, 1000000, 32
_NC, _NS, _L = 2, 16, 16
_NW = _NC * _NS


@functools.lru_cache(maxsize=None)
def _build(B, F, V, D, R, interpret=False):
    NW = _NW
    RW = B // NW
    NCHUNK = RW // R
    NIDX = R * F
    SLEN = 104 if NIDX % 104 == 0 else 8
    while NIDX % SLEN or SLEN > 128:
        SLEN -= 8
    NSTREAM = NIDX // SLEN
    L = _L

    mesh = plsc.VectorSubcoreMesh(
        core_axis_name="c", subcore_axis_name="s", num_cores=_NC,
        num_subcores=_NS)

    @functools.partial(
        pl.kernel,
        out_type=(
            jax.ShapeDtypeStruct((B * F,), jnp.float32),
            jax.ShapeDtypeStruct((B * D,), jnp.float32),
        ),
        mesh=mesh,
        scratch_types=[
            pltpu.VMEM((NIDX,), jnp.int32),
            pltpu.VMEM((NIDX,), jnp.float32),
            pltpu.VMEM((NIDX,), jnp.float32),
            pltpu.VMEM((NIDX, D), jnp.float32),
            pltpu.VMEM((NIDX,), jnp.float32),
            pltpu.VMEM((R * D,), jnp.float32),
            pltpu.SemaphoreType.DMA,
        ],
        compiler_params=pltpu.CompilerParams(
            needs_layout_passes=False, use_tc_tiling_on_sc=False),
        interpret=interpret,
    )
    def fm(fi, fv, em1, em2, y1, y2, idx_v, fv_v, w2_v, rows_v, y1_v, y2_v,
           sem):
        wid = lax.axis_index("s") * _NC + lax.axis_index("c")

        def chunk(c, carry):
            row0 = wid * RW + c * R
            flat0 = row0 * F
            pltpu.sync_copy(fi.at[pl.ds(flat0, NIDX)], idx_v)
            pltpu.sync_copy(fv.at[pl.ds(flat0, NIDX)], fv_v)
            handles = []
            for k in range(NSTREAM):
                sl = pl.ds(k * SLEN, SLEN)
                handles.append(
                    pltpu.async_copy(em1.at[idx_v.at[sl]], rows_v.at[sl], sem))
                handles.append(
                    pltpu.async_copy(em2.at[idx_v.at[sl]], w2_v.at[sl], sem))
            for h in handles:
                h.wait()

            for i in range(NIDX // L):
                s = pl.ds(i * L, L)
                y1_v[s] = w2_v[s] * fv_v[s]

            def row(b, _):
                fb = b * F
                z = jnp.zeros((L,), jnp.float32)
                a_lo, a_hi, s_lo, s_hi = z, z, z, z
                for f in range(F):
                    r = fb + f
                    fvf = plsc.load_gather(fv_v, [jnp.full((L,), r, jnp.int32)])
                    lo = rows_v[r, pl.ds(0, L)] * fvf
                    hi = rows_v[r, pl.ds(L, L)] * fvf
                    a_lo = a_lo + lo
                    a_hi = a_hi + hi
                    s_lo = s_lo + lo * lo
                    s_hi = s_hi + hi * hi
                y2_v[pl.ds(b * D, L)] = 0.5 * (a_lo * a_lo - s_lo)
                y2_v[pl.ds(b * D + L, L)] = 0.5 * (a_hi * a_hi - s_hi)
                return 0

            lax.fori_loop(0, R, row, 0, unroll=False)

            pltpu.sync_copy(y1_v, y1.at[pl.ds(flat0, NIDX)])
            pltpu.sync_copy(y2_v, y2.at[pl.ds(row0 * D, R * D)])
            return carry

        lax.fori_loop(0, NCHUNK, chunk, 0, unroll=False)

    return fm


def kernel(feat_index, feat_value, em1_table, em2_table):
    B, F = feat_index.shape
    V, D = em1_table.shape
    fi = feat_index.astype(jnp.int32).reshape(B * F)
    fv = feat_value.reshape(B * F)
    em2 = em2_table.reshape(V)
    fm = _build(B, F, V, D, 32)
    y1, y2 = fm(fi, fv, em1_table, em2)
    return y1.reshape(B, F), y2.reshape(B, D)
